# Initial kernel scaffold; baseline (speedup 1.0000x reference)
import jax
import jax.numpy as jnp
from jax.experimental import pallas as pl


def kernel(x, edge_index, W1, b1, W2, b2):
    raise NotImplementedError("write your pallas kernel here")


# speedup vs baseline: 27.1072x; 27.1072x over previous
import functools

import jax
import jax.numpy as jnp
from jax import lax
from jax.experimental import pallas as pl
from jax.experimental.pallas import tpu as pltpu
from jax.experimental.pallas import tpu_sc as plsc

N_NODES = 10000
NP = 10240
N_EDGES = 160000
D = 256
H = 128
NC = 2
NS = 16
EP = 163840
K = 128
NCH = EP // (NS * K)
CHD = EP // (NC * NS * K)
RPT = NP // NS
MB = 1000


def _mesh():
    return plsc.VectorSubcoreMesh(core_axis_name="c", subcore_axis_name="s",
                                  num_cores=NC, num_subcores=NS)


def _sc_degree(dstR, ones, z128):
    @functools.partial(
        pl.kernel,
        out_type=jax.ShapeDtypeStruct((NC * NP, H), jnp.float32),
        mesh=_mesh(),
        scratch_types=[
            pltpu.VMEM((CHD, K), jnp.int32),
            pltpu.VMEM((K, H), jnp.float32),
            pltpu.VMEM_SHARED((NP, H), jnp.float32),
            pltpu.SemaphoreType.DMA,
        ],
    )
    def deg_kernel(dstR_hbm, ones_hbm, z_hbm, deg_hbm, di_v, ones_v, acc_sh, sem):
        c = lax.axis_index("c")
        s = lax.axis_index("s")
        w = c * NS + s
        pltpu.sync_copy(dstR_hbm.at[pl.ds(w * CHD, CHD)], di_v)
        pltpu.sync_copy(ones_hbm, ones_v)
        pltpu.sync_copy(z_hbm, acc_sh.at[pl.ds(s * RPT, RPT)])
        plsc.subcore_barrier()

        def fire(i, carry):
            pltpu.async_copy(ones_v, acc_sh.at[di_v.at[i]], sem, add=True)
            return carry

        lax.fori_loop(0, CHD, fire, 0)

        def drain(i, carry):
            pltpu.make_async_copy(ones_v, acc_sh.at[di_v.at[i]], sem).wait()
            return carry

        lax.fori_loop(0, CHD, drain, 0)
        plsc.subcore_barrier()
        pltpu.sync_copy(acc_sh.at[pl.ds(s * RPT, RPT)],
                        deg_hbm.at[pl.ds(c * NP + s * RPT, RPT)])

    return deg_kernel(dstR, ones, z128)


def _sc_aggregate(g_flat, srcR, dstR, z128):
    @functools.partial(
        pl.kernel,
        out_type=jax.ShapeDtypeStruct((NC * NP, H), jnp.float32),
        mesh=_mesh(),
        scratch_types=[
            pltpu.VMEM((NCH, K), jnp.int32),
            pltpu.VMEM((NCH, K), jnp.int32),
            pltpu.VMEM((K, H), jnp.float32),
            pltpu.VMEM_SHARED((NP, H), jnp.float32),
            pltpu.SemaphoreType.DMA,
        ],
    )
    def agg_kernel(g_hbm, srcR_hbm, dstR_hbm, z_hbm, out_hbm,
                   si_v, di_v, rows_v, acc_sh, semA):
        c = lax.axis_index("c")
        s = lax.axis_index("s")
        w = c * NS + s
        pltpu.sync_copy(srcR_hbm.at[pl.ds(w * NCH, NCH)], si_v)
        pltpu.sync_copy(dstR_hbm.at[pl.ds(s * NCH, NCH)], di_v)
        pltpu.sync_copy(z_hbm, acc_sh.at[pl.ds(s * RPT, RPT)])
        plsc.subcore_barrier()

        @functools.partial(plsc.parallel_loop, 0, NCH, unroll=2)
        def _(p):
            pltpu.async_copy(g_hbm.at[si_v.at[p]], rows_v, semA).wait()
            pltpu.sync_copy(rows_v, acc_sh.at[di_v.at[p]], add=True)
        plsc.subcore_barrier()
        pltpu.sync_copy(acc_sh.at[pl.ds(s * RPT, RPT)],
                        out_hbm.at[pl.ds(c * NP + s * RPT, RPT)])

    return agg_kernel(g_flat, srcR, dstR, z128)


def _tc_mm(x, W1):
    def body(x_ref, w_ref, o_ref):
        o_ref[0] = jnp.dot(x_ref[...], w_ref[...],
                           preferred_element_type=jnp.float32)

    return pl.pallas_call(
        body,
        grid=(N_NODES // MB, NC),
        in_specs=[
            pl.BlockSpec((MB, D), lambda i, j: (i, 0)),
            pl.BlockSpec((D, H), lambda i, j: (0, j)),
        ],
        out_specs=pl.BlockSpec((1, MB, H), lambda i, j: (j, i, 0)),
        out_shape=jax.ShapeDtypeStruct((NC, N_NODES, H), jnp.float32),
    )(x, W1)


def _tc_scale(h, d0, d1):
    def body(h_ref, da_ref, db_ref, o_ref):
        dinv = lax.rsqrt(da_ref[:, 0:1] + db_ref[:, 0:1] + 1.0)
        o_ref[0] = dinv * h_ref[0]

    return pl.pallas_call(
        body,
        grid=(N_NODES // MB, NC),
        in_specs=[
            pl.BlockSpec((1, MB, H), lambda i, j: (j, i, 0)),
            pl.BlockSpec((MB, 16), lambda i, j: (i, 0)),
            pl.BlockSpec((MB, 16), lambda i, j: (i, 0)),
        ],
        out_specs=pl.BlockSpec((1, MB, H), lambda i, j: (j, i, 0)),
        out_shape=jax.ShapeDtypeStruct((NC, N_NODES, H), jnp.float32),
    )(h, d0, d1)


def _tc2_u(g1, d0, d1, b1, W2):
    def body(ga, gb, da_ref, db_ref, b_ref, w_ref, o_ref):
        dinv = lax.rsqrt(da_ref[:, 0:1] + db_ref[:, 0:1] + 1.0)
        y = dinv * jnp.concatenate([ga[0], gb[0]], axis=1) + b_ref[...]
        o_ref[0] = jnp.dot(y, w_ref[...], preferred_element_type=jnp.float32)

    return pl.pallas_call(
        body,
        grid=(N_NODES // MB, NC),
        in_specs=[
            pl.BlockSpec((1, MB, H), lambda i, j: (0, i, 0)),
            pl.BlockSpec((1, MB, H), lambda i, j: (1, i, 0)),
            pl.BlockSpec((MB, 16), lambda i, j: (i, 0)),
            pl.BlockSpec((MB, 16), lambda i, j: (i, 0)),
            pl.BlockSpec((1, D), lambda i, j: (0, 0)),
            pl.BlockSpec((D, H), lambda i, j: (0, j)),
        ],
        out_specs=pl.BlockSpec((1, MB, H), lambda i, j: (j, i, 0)),
        out_shape=jax.ShapeDtypeStruct((NC, N_NODES, H), jnp.float32),
    )(g1, g1, d0, d1, b1, W2)


def _tc2_s(s1a, s1b, u, d0, d1, W2):
    def body(sa, sb, ua, da_ref, db_ref, w_ref, o_ref):
        dinv = lax.rsqrt(da_ref[:, 0:1] + db_ref[:, 0:1] + 1.0)
        t = dinv * jnp.concatenate([sa[...], sb[...]], axis=1)
        o_ref[0] = dinv * (jnp.dot(t, w_ref[...],
                                   preferred_element_type=jnp.float32) + ua[0])

    return pl.pallas_call(
        body,
        grid=(N_NODES // MB, NC),
        in_specs=[
            pl.BlockSpec((MB, H), lambda i, j: (i, 0)),
            pl.BlockSpec((MB, H), lambda i, j: (i, 0)),
            pl.BlockSpec((1, MB, H), lambda i, j: (j, i, 0)),
            pl.BlockSpec((MB, 16), lambda i, j: (i, 0)),
            pl.BlockSpec((MB, 16), lambda i, j: (i, 0)),
            pl.BlockSpec((D, H), lambda i, j: (0, j)),
        ],
        out_specs=pl.BlockSpec((1, MB, H), lambda i, j: (j, i, 0)),
        out_shape=jax.ShapeDtypeStruct((NC, N_NODES, H), jnp.float32),
    )(s1a, s1b, u, d0, d1, W2)


def _tc3_v(g2, d0, d1, b2h):
    def body(g_ref, da_ref, db_ref, b_ref, o_ref):
        dinv = lax.rsqrt(da_ref[:, 0:1] + db_ref[:, 0:1] + 1.0)
        o_ref[0] = dinv * g_ref[0] + b_ref[0]

    return pl.pallas_call(
        body,
        grid=(N_NODES // MB, NC),
        in_specs=[
            pl.BlockSpec((1, MB, H), lambda i, j: (j, i, 0)),
            pl.BlockSpec((MB, 16), lambda i, j: (i, 0)),
            pl.BlockSpec((MB, 16), lambda i, j: (i, 0)),
            pl.BlockSpec((1, 1, H), lambda i, j: (j, 0, 0)),
        ],
        out_specs=pl.BlockSpec((1, MB, H), lambda i, j: (j, i, 0)),
        out_shape=jax.ShapeDtypeStruct((NC, N_NODES, H), jnp.float32),
    )(g2, d0, d1, b2h)


def _tc3_s(s2a, s2b, v, d0, d1):
    def body(sa, sb, va, vb, da_ref, db_ref, o_ref):
        dinv = lax.rsqrt(da_ref[:, 0:1] + db_ref[:, 0:1] + 1.0)
        y = jnp.concatenate([dinv * sa[...] + va[0],
                             dinv * sb[...] + vb[0]], axis=1)
        o_ref[...] = y

    return pl.pallas_call(
        body,
        grid=(N_NODES // MB,),
        in_specs=[
            pl.BlockSpec((MB, H), lambda i: (i, 0)),
            pl.BlockSpec((MB, H), lambda i: (i, 0)),
            pl.BlockSpec((1, MB, H), lambda i: (0, i, 0)),
            pl.BlockSpec((1, MB, H), lambda i: (1, i, 0)),
            pl.BlockSpec((MB, 16), lambda i: (i, 0)),
            pl.BlockSpec((MB, 16), lambda i: (i, 0)),
        ],
        out_specs=pl.BlockSpec((MB, D), lambda i: (i, 0)),
        out_shape=jax.ShapeDtypeStruct((N_NODES, D), jnp.float32),
    )(s2a, s2b, v, v, d0, d1)


def kernel(x, edge_index, W1, b1, W2, b2):
    src = edge_index[0].astype(jnp.int32)
    dst = edge_index[1].astype(jnp.int32)
    pad = jnp.arange(EP - N_EDGES, dtype=jnp.int32)
    src_p = jnp.concatenate([src, pad % N_NODES])
    dst_p = jnp.concatenate([dst, N_NODES + pad % (NP - N_NODES)])
    dstR = dst_p.reshape(NS * NCH, K)
    srcR = jnp.concatenate([src_p, src_p + N_NODES]).reshape(NC * NS * NCH, K)
    ones = jnp.ones((K, H), jnp.float32)
    z128 = jnp.zeros((RPT, H), jnp.float32)
    b1r = b1.reshape(1, D).astype(jnp.float32)
    b2h = b2.reshape(NC, 1, H).astype(jnp.float32)

    degout = _sc_degree(dstR, ones, z128).reshape(NC, NP, H)
    h1 = _tc_mm(x, W1)
    d0 = degout[0, :N_NODES, :16]
    d1 = degout[1, :N_NODES, :16]
    g1 = _tc_scale(h1, d0, d1)
    s1 = _sc_aggregate(g1.reshape(NC * N_NODES, H), srcR, dstR, z128)
    u = _tc2_u(g1, d0, d1, b1r, W2)
    s1 = s1.reshape(NC, NP, H)[:, :N_NODES]
    g2 = _tc2_s(s1[0], s1[1], u, d0, d1, W2)
    s2 = _sc_aggregate(g2.reshape(NC * N_NODES, H), srcR, dstR, z128)
    v = _tc3_v(g2, d0, d1, b2h)
    s2 = s2.reshape(NC, NP, H)[:, :N_NODES]
    return _tc3_s(s2[0], s2[1], v, d0, d1)
